# baseline (device time: 508796 ns/iter reference)
import jax
import jax.numpy as jnp
from jax import lax
from jax.experimental import pallas as pl
from jax.experimental.pallas import tpu as pltpu

B = 16
H = 16
D = 64
HB = 8
SCALE = D ** -0.5
CW = D + 2


def kernel(Q, K, V):
    kv_local = K.shape[1]

    def body(q_ref, k_ref, v_ref, out_ref, send_ref, recv_ref,
             send_sem, recv_sem):
        b = pl.program_id(0)
        hg = pl.program_id(1)
        nb = pl.num_programs(0)
        nhg = pl.num_programs(1)

        qall = q_ref[b, 0, pl.ds(hg * HB, HB), :]
        qall = qall.astype(jnp.bfloat16)
        o_l, m_l, l_l = [], [], []
        for h0 in range(HB):
            k2 = k_ref[0, :, h0, :].astype(jnp.bfloat16)
            v2 = v_ref[0, :, h0, :].astype(jnp.bfloat16)
            q2 = qall[h0:h0 + 1, :]
            s = lax.dot_general(
                q2, k2, (((1,), (1,)), ((), ())),
                preferred_element_type=jnp.float32,
            ) * SCALE
            m = jnp.max(s, axis=1, keepdims=True)
            p = jnp.exp(s - m)
            l = jnp.sum(p, axis=1, keepdims=True)
            o = lax.dot_general(
                p.astype(jnp.bfloat16), v2, (((1,), (0,)), ((), ())),
                preferred_element_type=jnp.float32,
            )
            o_l.append(o)
            m_l.append(m)
            l_l.append(l)
        row = jnp.concatenate(
            [jnp.concatenate(o_l, axis=0),
             jnp.concatenate(m_l, axis=0),
             jnp.concatenate(l_l, axis=0)],
            axis=1,
        )
        send_ref[b, pl.ds(hg * HB, HB), pl.ds(0, CW)] = row

        @pl.when((b == nb - 1) & (hg == nhg - 1))
        def _():
            my_x = lax.axis_index("x")
            my_y = lax.axis_index("y")
            my_z = lax.axis_index("z")
            nbr = (1 - my_x, my_y, my_z)

            barrier = pltpu.get_barrier_semaphore()
            pl.semaphore_signal(
                barrier, inc=1, device_id=nbr,
                device_id_type=pl.DeviceIdType.MESH,
            )
            pl.semaphore_wait(barrier, 1)

            rdma = pltpu.make_async_remote_copy(
                src_ref=send_ref,
                dst_ref=recv_ref,
                send_sem=send_sem,
                recv_sem=recv_sem,
                device_id=nbr,
                device_id_type=pl.DeviceIdType.MESH,
            )
            rdma.start()
            rdma.wait()

            o_a = send_ref[:, :, 0:D]
            m_a = send_ref[:, :, D:D + 1]
            l_a = send_ref[:, :, D + 1:D + 2]
            o_b = recv_ref[:, :, 0:D]
            m_b = recv_ref[:, :, D:D + 1]
            l_b = recv_ref[:, :, D + 1:D + 2]
            m_n = jnp.maximum(m_a, m_b)
            alpha = jnp.exp(m_a - m_n)
            beta = jnp.exp(m_b - m_n)
            l_n = l_a * alpha + l_b * beta
            out_ref[:, 0, :, :] = (o_a * alpha + o_b * beta) / l_n

    return pl.pallas_call(
        body,
        grid=(B, H // HB),
        in_specs=[
            pl.BlockSpec((B, 1, H, D), lambda b, hg: (0, 0, 0, 0)),
            pl.BlockSpec((1, kv_local, HB, D), lambda b, hg: (b, 0, hg, 0)),
            pl.BlockSpec((1, kv_local, HB, D), lambda b, hg: (b, 0, hg, 0)),
        ],
        out_specs=pl.BlockSpec((B, 1, H, D), lambda b, hg: (0, 0, 0, 0)),
        out_shape=jax.ShapeDtypeStruct((B, 1, H, D), jnp.float32),
        scratch_shapes=[
            pltpu.VMEM((B, H, CW), jnp.float32),
            pltpu.VMEM((B, H, CW), jnp.float32),
            pltpu.SemaphoreType.DMA,
            pltpu.SemaphoreType.DMA,
        ],
        compiler_params=pltpu.CompilerParams(collective_id=0),
    )(Q, K, V)


# device time: 377010 ns/iter; 1.3496x vs baseline; 1.3496x over previous
import jax
import jax.numpy as jnp
from jax import lax
from jax.experimental import pallas as pl
from jax.experimental.pallas import tpu as pltpu

B = 16
H = 16
D = 64
HB = 8
SCALE = D ** -0.5
CW = D + 2


def kernel(Q, K, V):
    kv_local = K.shape[1]

    def body(q_ref, k_ref, v_ref, out_ref, send_ref, recv_ref,
             send_sem, recv_sem):
        b = pl.program_id(0)
        hg = pl.program_id(1)
        nb = pl.num_programs(0)
        nhg = pl.num_programs(1)

        qall = q_ref[b, 0, pl.ds(hg * HB, HB), :]
        o_l, m_l, l_l = [], [], []
        for h0 in range(HB):
            k2 = k_ref[0, :, h0, :]
            v2 = v_ref[0, :, h0, :]
            q2 = qall[h0:h0 + 1, :]
            s = lax.dot_general(
                q2, k2, (((1,), (1,)), ((), ())),
                preferred_element_type=jnp.float32,
            ) * SCALE
            m = jnp.max(s, axis=1, keepdims=True)
            p = jnp.exp(s - m)
            l = jnp.sum(p, axis=1, keepdims=True)
            o = lax.dot_general(
                p, v2, (((1,), (0,)), ((), ())),
                preferred_element_type=jnp.float32,
            )
            o_l.append(o)
            m_l.append(m)
            l_l.append(l)
        row = jnp.concatenate(
            [jnp.concatenate(o_l, axis=0),
             jnp.concatenate(m_l, axis=0),
             jnp.concatenate(l_l, axis=0)],
            axis=1,
        )
        send_ref[b, pl.ds(hg * HB, HB), pl.ds(0, CW)] = row

        @pl.when((b == nb - 1) & (hg == nhg - 1))
        def _():
            my_x = lax.axis_index("x")
            my_y = lax.axis_index("y")
            my_z = lax.axis_index("z")
            nbr = (1 - my_x, my_y, my_z)

            barrier = pltpu.get_barrier_semaphore()
            pl.semaphore_signal(
                barrier, inc=1, device_id=nbr,
                device_id_type=pl.DeviceIdType.MESH,
            )
            pl.semaphore_wait(barrier, 1)

            rdma = pltpu.make_async_remote_copy(
                src_ref=send_ref,
                dst_ref=recv_ref,
                send_sem=send_sem,
                recv_sem=recv_sem,
                device_id=nbr,
                device_id_type=pl.DeviceIdType.MESH,
            )
            rdma.start()
            rdma.wait()

            o_a = send_ref[:, :, 0:D]
            m_a = send_ref[:, :, D:D + 1]
            l_a = send_ref[:, :, D + 1:D + 2]
            o_b = recv_ref[:, :, 0:D]
            m_b = recv_ref[:, :, D:D + 1]
            l_b = recv_ref[:, :, D + 1:D + 2]
            m_n = jnp.maximum(m_a, m_b)
            alpha = jnp.exp(m_a - m_n)
            beta = jnp.exp(m_b - m_n)
            l_n = l_a * alpha + l_b * beta
            out_ref[:, 0, :, :] = (o_a * alpha + o_b * beta) / l_n

    return pl.pallas_call(
        body,
        grid=(B, H // HB),
        in_specs=[
            pl.BlockSpec((B, 1, H, D), lambda b, hg: (0, 0, 0, 0)),
            pl.BlockSpec((1, kv_local, HB, D), lambda b, hg: (b, 0, hg, 0)),
            pl.BlockSpec((1, kv_local, HB, D), lambda b, hg: (b, 0, hg, 0)),
        ],
        out_specs=pl.BlockSpec((B, 1, H, D), lambda b, hg: (0, 0, 0, 0)),
        out_shape=jax.ShapeDtypeStruct((B, 1, H, D), jnp.float32),
        scratch_shapes=[
            pltpu.VMEM((B, H, CW), jnp.float32),
            pltpu.VMEM((B, H, CW), jnp.float32),
            pltpu.SemaphoreType.DMA,
            pltpu.SemaphoreType.DMA,
        ],
        compiler_params=pltpu.CompilerParams(collective_id=0),
    )(Q, K, V)


# device time: 301569 ns/iter; 1.6872x vs baseline; 1.2502x over previous
import jax
import jax.numpy as jnp
from jax import lax
from jax.experimental import pallas as pl
from jax.experimental.pallas import tpu as pltpu

B = 16
H = 16
D = 64
SCALE = D ** -0.5
CW = D + 2
NC = 2


def kernel(Q, K, V):
    kv_local = K.shape[1]
    rows = kv_local * H // NC
    Kr = K.reshape(B, kv_local * H, D)
    Vr = V.reshape(B, kv_local * H, D)

    def body(q_ref, k_ref, v_ref, out_ref, send_ref, recv_ref,
             send_sem, recv_sem):
        b = pl.program_id(0)
        c = pl.program_id(1)
        nb = pl.num_programs(0)
        nc = pl.num_programs(1)

        qall = q_ref[b, 0, :, :]
        k2d = k_ref[0]
        v2d = v_ref[0]

        s = lax.dot_general(
            qall, k2d, (((1,), (1,)), ((), ())),
            preferred_element_type=jnp.float32,
        ) * SCALE
        lane = lax.broadcasted_iota(jnp.int32, (H, rows), 1)
        head = lax.broadcasted_iota(jnp.int32, (H, rows), 0)
        s = jnp.where(jnp.bitwise_and(lane, H - 1) == head, s, -1e30)
        m = jnp.max(s, axis=1, keepdims=True)
        p = jnp.exp(s - m)
        l = jnp.sum(p, axis=1, keepdims=True)
        o = lax.dot_general(
            p, v2d, (((1,), (0,)), ((), ())),
            preferred_element_type=jnp.float32,
        )
        row = jnp.concatenate([o, m, l], axis=1)

        @pl.when(c == 0)
        def _():
            send_ref[b, :, pl.ds(0, CW)] = row

        @pl.when(c > 0)
        def _():
            prev = send_ref[b, :, pl.ds(0, CW)]
            o0 = prev[:, 0:D]
            m0 = prev[:, D:D + 1]
            l0 = prev[:, D + 1:D + 2]
            m01 = jnp.maximum(m0, m)
            a0 = jnp.exp(m0 - m01)
            a1 = jnp.exp(m - m01)
            send_ref[b, :, pl.ds(0, CW)] = jnp.concatenate(
                [o0 * a0 + o * a1, m01, l0 * a0 + l * a1], axis=1)

        @pl.when((b == nb - 1) & (c == nc - 1))
        def _():
            my_x = lax.axis_index("x")
            my_y = lax.axis_index("y")
            my_z = lax.axis_index("z")
            nbr = (1 - my_x, my_y, my_z)

            barrier = pltpu.get_barrier_semaphore()
            pl.semaphore_signal(
                barrier, inc=1, device_id=nbr,
                device_id_type=pl.DeviceIdType.MESH,
            )
            pl.semaphore_wait(barrier, 1)

            rdma = pltpu.make_async_remote_copy(
                src_ref=send_ref,
                dst_ref=recv_ref,
                send_sem=send_sem,
                recv_sem=recv_sem,
                device_id=nbr,
                device_id_type=pl.DeviceIdType.MESH,
            )
            rdma.start()
            rdma.wait()

            o_a = send_ref[:, :, 0:D]
            m_a = send_ref[:, :, D:D + 1]
            l_a = send_ref[:, :, D + 1:D + 2]
            o_b = recv_ref[:, :, 0:D]
            m_b = recv_ref[:, :, D:D + 1]
            l_b = recv_ref[:, :, D + 1:D + 2]
            m_n = jnp.maximum(m_a, m_b)
            alpha = jnp.exp(m_a - m_n)
            beta = jnp.exp(m_b - m_n)
            l_n = l_a * alpha + l_b * beta
            out_ref[:, 0, :, :] = (o_a * alpha + o_b * beta) / l_n

    return pl.pallas_call(
        body,
        grid=(B, NC),
        in_specs=[
            pl.BlockSpec((B, 1, H, D), lambda b, c: (0, 0, 0, 0)),
            pl.BlockSpec((1, rows, D), lambda b, c: (b, c, 0)),
            pl.BlockSpec((1, rows, D), lambda b, c: (b, c, 0)),
        ],
        out_specs=pl.BlockSpec((B, 1, H, D), lambda b, c: (0, 0, 0, 0)),
        out_shape=jax.ShapeDtypeStruct((B, 1, H, D), jnp.float32),
        scratch_shapes=[
            pltpu.VMEM((B, H, CW), jnp.float32),
            pltpu.VMEM((B, H, CW), jnp.float32),
            pltpu.SemaphoreType.DMA,
            pltpu.SemaphoreType.DMA,
        ],
        compiler_params=pltpu.CompilerParams(collective_id=0),
    )(Q, Kr, Vr)
